# baseline (speedup 1.0000x reference)
import jax
import jax.numpy as jnp
from jax import lax
from jax.experimental import pallas as pl
from jax.experimental.pallas import tpu as pltpu
from jax.experimental.pallas import tpu_sc as plsc

NC = 2
NS = 16
NW = NC * NS
C = 20
ZR = 16
FW = 128
NB = 10

EPS = 1e-5


def _pad_rows(n):
  rps = (-(-n // NS) + ZR - 1) // ZR * ZR
  return rps, rps * NS


def _sc_aggregate(y, idx3):
  np_, f = y.shape
  assert f == FW
  nw_, nb, two, nb_, c_ = idx3.shape
  assert (nw_, two, nb_, c_) == (NW, 2, NB, C) and nb % 2 == 0
  rps = np_ // NS
  nz = rps // ZR
  assert rps * NS == np_ and nz * ZR == rps

  mesh = plsc.VectorSubcoreMesh(core_axis_name="c", subcore_axis_name="s",
                                num_cores=NC, num_subcores=NS)

  def body(y_h, idx_h, out_h, ib0, ib1, rows, zbuf, acc, semg, sems, semi, semz):
    c = lax.axis_index("c")
    s = lax.axis_index("s")
    tile = s * NC + c

    @pl.loop(0, ZR)
    def _(r):
      @pl.loop(0, f, step=16)
      def _(j):
        zbuf.at[r, pl.ds(j, 16)][...] = jnp.zeros((16,), jnp.float32)

    base = s * rps
    zd = [pltpu.async_copy(zbuf, acc.at[pl.ds(base + i * ZR, ZR)], semz)
          for i in range(nz)]
    pltpu.async_copy(idx_h.at[tile, 0], ib0, semi)
    for d in zd:
      d.wait()
    plsc.subcore_barrier()

    def run_block(b, cur, nxt):
      pltpu.make_async_copy(idx_h.at[tile, 0], cur, semi).wait()
      pltpu.async_copy(idx_h.at[tile, (b + 1) % nb], nxt, semi)
      gd = [pltpu.async_copy(y_h.at[cur.at[0, k]], rows.at[k], semg)
            for k in range(NB)]
      sd = []
      for k in range(NB):
        gd[k].wait()
        sd.append(pltpu.async_copy(rows.at[k], acc.at[cur.at[1, k]],
                                   sems, add=True))
      for k in range(NB):
        sd[k].wait()

    @pl.loop(0, nb // 2)
    def _(j):
      run_block(2 * j, ib0, ib1)
      run_block(2 * j + 1, ib1, ib0)
    pltpu.make_async_copy(idx_h.at[tile, 0], ib0, semi).wait()
    plsc.subcore_barrier()

    pltpu.sync_copy(acc.at[pl.ds(base, rps)], out_h.at[c, pl.ds(base, rps)])

  k = pl.kernel(
      body,
      out_type=jax.ShapeDtypeStruct((NC, np_, f), jnp.float32),
      mesh=mesh,
      scratch_types=[
          pltpu.VMEM((2, NB, C), jnp.int32),
          pltpu.VMEM((2, NB, C), jnp.int32),
          pltpu.VMEM((NB, C, f), jnp.float32),
          pltpu.VMEM((ZR, f), jnp.float32),
          pltpu.VMEM_SHARED((np_, f), jnp.float32),
          pltpu.SemaphoreType.DMA,
          pltpu.SemaphoreType.DMA,
          pltpu.SemaphoreType.DMA,
          pltpu.SemaphoreType.DMA,
      ])
  return k(y, idx3)


def _tc_project(x, wT, with_ones):
  n = x.shape[0]
  m = wT.shape[1]
  _, np_ = _pad_rows(n)

  def body(x_ref, w_ref, o_ref):
    yv = jnp.dot(x_ref[...], w_ref[...], preferred_element_type=jnp.float32)
    extra = jnp.full((n, FW - m), 1.0 if with_ones else 0.0, jnp.float32)
    if with_ones:
      extra = extra * (jnp.arange(FW - m)[None, :] < 1).astype(jnp.float32)
    o_ref[:n] = jnp.concatenate([yv, extra], axis=1)
    o_ref[n:] = jnp.zeros((np_ - n, FW), jnp.float32)

  return pl.pallas_call(
      body,
      out_shape=jax.ShapeDtypeStruct((np_, FW), jnp.float32),
  )(x, wT)


def _tc_combine(h, wlT, b2, sp, invd, g2, be2, wnT):
  n = h.shape[0]
  m = wlT.shape[1]
  _, np_ = _pad_rows(n)
  first = invd is None

  def body(*refs):
    if first:
      (h_ref, wl_ref, b_ref, sp_ref, g_ref, be_ref, wn_ref,
       h_out, y_out, iv_out) = refs
    else:
      (h_ref, wl_ref, b_ref, sp_ref, iv_ref, g_ref, be_ref, wn_ref,
       h_out, y_out) = refs
    ssum = sp_ref[0, :n] + sp_ref[1, :n]
    if first:
      iv = 1.0 / jnp.maximum(ssum[:, m:m + 1], 1.0)
      iv_out[...] = iv
    else:
      iv = iv_ref[...]
    t = (jnp.dot(h_ref[...], wl_ref[...], preferred_element_type=jnp.float32)
         + b_ref[...] + ssum[:, :m] * iv)
    mean = jnp.mean(t, axis=0, keepdims=True)
    var = jnp.mean((t - mean) ** 2, axis=0, keepdims=True)
    xn = (t - mean) / jnp.sqrt(var + EPS)
    hn = jnp.maximum(xn * g_ref[...] + be_ref[...], 0.0)
    h_out[...] = hn
    yn = jnp.dot(hn, wn_ref[...], preferred_element_type=jnp.float32)
    y_out[:n] = jnp.concatenate(
        [yn, jnp.zeros((n, FW - wnT.shape[1]), jnp.float32)], axis=1)
    y_out[n:] = jnp.zeros((np_ - n, FW), jnp.float32)

  out_shape = [jax.ShapeDtypeStruct((n, m), jnp.float32),
               jax.ShapeDtypeStruct((np_, FW), jnp.float32)]
  if first:
    out_shape.append(jax.ShapeDtypeStruct((n, 1), jnp.float32))
  args = (h, wlT, b2, sp, g2, be2, wnT) if first else (
      h, wlT, b2, sp, invd, g2, be2, wnT)
  return pl.pallas_call(body, out_shape=out_shape)(*args)


def _tc_final(h, wlT, b2, sp, invd, g2, be2, hw1T, hb1, hw2T, hb2):
  n = h.shape[0]
  m = wlT.shape[1]

  def body(h_ref, wl_ref, b_ref, sp_ref, iv_ref, g_ref, be_ref,
           w1_ref, b1_ref, w2_ref, bb2_ref, o_ref):
    ssum = sp_ref[0, :n] + sp_ref[1, :n]
    t = (jnp.dot(h_ref[...], wl_ref[...], preferred_element_type=jnp.float32)
         + b_ref[...] + ssum[:, :m] * iv_ref[...])
    mean = jnp.mean(t, axis=0, keepdims=True)
    var = jnp.mean((t - mean) ** 2, axis=0, keepdims=True)
    xn = (t - mean) / jnp.sqrt(var + EPS)
    hn = jnp.maximum(xn * g_ref[...] + be_ref[...], 0.0)
    t1 = jnp.maximum(
        jnp.dot(hn, w1_ref[...], preferred_element_type=jnp.float32)
        + b1_ref[...], 0.0)
    o_ref[...] = (jnp.dot(t1, w2_ref[...], preferred_element_type=jnp.float32)
                  + bb2_ref[...])

  return pl.pallas_call(
      body, out_shape=jax.ShapeDtypeStruct((n, 1), jnp.float32),
  )(h, wlT, b2, sp, invd, g2, be2, hw1T, hb1, hw2T, hb2)


def kernel(x, edge_index, batch, W0, b0, g0, be0, W1, b1, g1, be1,
           W2, b2, g2, be2, Hw1, Hb1, Hw2, Hb2):
  del batch
  n, d = x.shape
  hd = W0.shape[0]
  srct = edge_index[0].reshape(NW, -1, NB, C)
  dstt = edge_index[1].reshape(NW, -1, NB, C)
  idx3 = jnp.stack([srct, dstt], axis=2)

  w0l, w0r = W0[:, :d].T, W0[:, d:].T
  w1l, w1r = W1[:, :hd].T, W1[:, hd:].T
  w2l, w2r = W2[:, :hd].T, W2[:, hd:].T
  r2 = lambda v: v.reshape(1, -1)

  y0 = _tc_project(x, w0r, with_ones=True)
  s0 = _sc_aggregate(y0, idx3)
  h1, y1, invd = _tc_combine(x, w0l, r2(b0), s0, None, r2(g0), r2(be0), w1r)
  s1 = _sc_aggregate(y1, idx3)
  h2, y2 = _tc_combine(h1, w1l, r2(b1), s1, invd, r2(g1), r2(be1), w2r)
  s2 = _sc_aggregate(y2, idx3)
  out = _tc_final(h2, w2l, r2(b2), s2, invd, r2(g2), r2(be2),
                  Hw1.T, r2(Hb1), Hw2.T, r2(Hb2))
  return out


# speedup vs baseline: 1.1160x; 1.1160x over previous
import jax
import jax.numpy as jnp
from jax import lax
from jax.experimental import pallas as pl
from jax.experimental.pallas import tpu as pltpu
from jax.experimental.pallas import tpu_sc as plsc

NC = 2
NS = 16
NW = NC * NS
C = 40
ZR = 16
FW = 128
NB = 5

EPS = 1e-5


def _pad_rows(n):
  rps = (-(-n // NS) + ZR - 1) // ZR * ZR
  return rps, rps * NS


def _sc_aggregate(y, idx3):
  np_, f = y.shape
  assert f == FW
  nw_, nb, two, nb_, c_ = idx3.shape
  assert (nw_, two, nb_, c_) == (NW, 2, NB, C) and nb % 2 == 0
  rps = np_ // NS
  nz = rps // ZR
  assert rps * NS == np_ and nz * ZR == rps

  mesh = plsc.VectorSubcoreMesh(core_axis_name="c", subcore_axis_name="s",
                                num_cores=NC, num_subcores=NS)

  def body(y_h, idx_h, out_h, iba, ibb, rows, zbuf, acc,
           semg, sems, semia, semib, semz):
    c = lax.axis_index("c")
    s = lax.axis_index("s")
    tile = s * NC + c

    @pl.loop(0, ZR)
    def _(r):
      @pl.loop(0, f, step=16)
      def _(j):
        zbuf.at[r, pl.ds(j, 16)][...] = jnp.zeros((16,), jnp.float32)

    base = s * rps
    zd = [pltpu.async_copy(zbuf, acc.at[pl.ds(base + i * ZR, ZR)], semz)
          for i in range(nz)]
    pltpu.async_copy(idx_h.at[tile, 0], iba, semia).wait()
    pltpu.async_copy(idx_h.at[tile, 1], ibb, semib)
    for k in range(NB):
      pltpu.async_copy(y_h.at[iba.at[0, k]], rows.at[k], semg)
    for d in zd:
      d.wait()
    plsc.subcore_barrier()

    def run_block(b, cur, nxt, semcur, semnxt):
      sd = []
      for k in range(NB):
        pltpu.make_async_copy(y_h.at[cur.at[0, k]], rows.at[k], semg).wait()
        sd.append(pltpu.async_copy(rows.at[k], acc.at[cur.at[1, k]],
                                   sems, add=True))
      pltpu.make_async_copy(idx_h.at[tile, 0], nxt, semnxt).wait()
      for k in range(NB):
        sd[k].wait()
        pltpu.async_copy(y_h.at[nxt.at[0, k]], rows.at[k], semg)
      pltpu.async_copy(idx_h.at[tile, (b + 2) % nb], cur, semcur)

    @pl.loop(0, nb // 2)
    def _(j):
      run_block(2 * j, iba, ibb, semia, semib)
      run_block(2 * j + 1, ibb, iba, semib, semia)
    for k in range(NB):
      pltpu.make_async_copy(y_h.at[iba.at[0, k]], rows.at[k], semg).wait()
    pltpu.make_async_copy(idx_h.at[tile, 0], ibb, semib).wait()
    plsc.subcore_barrier()

    pltpu.sync_copy(acc.at[pl.ds(base, rps)], out_h.at[c, pl.ds(base, rps)])

  k = pl.kernel(
      body,
      out_type=jax.ShapeDtypeStruct((NC, np_, f), jnp.float32),
      mesh=mesh,
      scratch_types=[
          pltpu.VMEM((2, NB, C), jnp.int32),
          pltpu.VMEM((2, NB, C), jnp.int32),
          pltpu.VMEM((NB, C, f), jnp.float32),
          pltpu.VMEM((ZR, f), jnp.float32),
          pltpu.VMEM_SHARED((np_, f), jnp.float32),
          pltpu.SemaphoreType.DMA,
          pltpu.SemaphoreType.DMA,
          pltpu.SemaphoreType.DMA,
          pltpu.SemaphoreType.DMA,
          pltpu.SemaphoreType.DMA,
      ])
  return k(y, idx3)


def _tc_project(x, wT, with_ones):
  n = x.shape[0]
  m = wT.shape[1]
  _, np_ = _pad_rows(n)

  def body(x_ref, w_ref, o_ref):
    yv = jnp.dot(x_ref[...], w_ref[...], preferred_element_type=jnp.float32)
    extra = jnp.full((n, FW - m), 1.0 if with_ones else 0.0, jnp.float32)
    if with_ones:
      extra = extra * (jnp.arange(FW - m)[None, :] < 1).astype(jnp.float32)
    o_ref[:n] = jnp.concatenate([yv, extra], axis=1)
    o_ref[n:] = jnp.zeros((np_ - n, FW), jnp.float32)

  return pl.pallas_call(
      body,
      out_shape=jax.ShapeDtypeStruct((np_, FW), jnp.float32),
  )(x, wT)


def _tc_combine(h, wlT, b2, sp, invd, g2, be2, wnT):
  n = h.shape[0]
  m = wlT.shape[1]
  _, np_ = _pad_rows(n)
  first = invd is None

  def body(*refs):
    if first:
      (h_ref, wl_ref, b_ref, sp_ref, g_ref, be_ref, wn_ref,
       h_out, y_out, iv_out) = refs
    else:
      (h_ref, wl_ref, b_ref, sp_ref, iv_ref, g_ref, be_ref, wn_ref,
       h_out, y_out) = refs
    ssum = sp_ref[0, :n] + sp_ref[1, :n]
    if first:
      iv = 1.0 / jnp.maximum(ssum[:, m:m + 1], 1.0)
      iv_out[...] = iv
    else:
      iv = iv_ref[...]
    t = (jnp.dot(h_ref[...], wl_ref[...], preferred_element_type=jnp.float32)
         + b_ref[...] + ssum[:, :m] * iv)
    mean = jnp.mean(t, axis=0, keepdims=True)
    var = jnp.mean((t - mean) ** 2, axis=0, keepdims=True)
    xn = (t - mean) / jnp.sqrt(var + EPS)
    hn = jnp.maximum(xn * g_ref[...] + be_ref[...], 0.0)
    h_out[...] = hn
    yn = jnp.dot(hn, wn_ref[...], preferred_element_type=jnp.float32)
    y_out[:n] = jnp.concatenate(
        [yn, jnp.zeros((n, FW - wnT.shape[1]), jnp.float32)], axis=1)
    y_out[n:] = jnp.zeros((np_ - n, FW), jnp.float32)

  out_shape = [jax.ShapeDtypeStruct((n, m), jnp.float32),
               jax.ShapeDtypeStruct((np_, FW), jnp.float32)]
  if first:
    out_shape.append(jax.ShapeDtypeStruct((n, 1), jnp.float32))
  args = (h, wlT, b2, sp, g2, be2, wnT) if first else (
      h, wlT, b2, sp, invd, g2, be2, wnT)
  return pl.pallas_call(body, out_shape=out_shape)(*args)


def _tc_final(h, wlT, b2, sp, invd, g2, be2, hw1T, hb1, hw2T, hb2):
  n = h.shape[0]
  m = wlT.shape[1]

  def body(h_ref, wl_ref, b_ref, sp_ref, iv_ref, g_ref, be_ref,
           w1_ref, b1_ref, w2_ref, bb2_ref, o_ref):
    ssum = sp_ref[0, :n] + sp_ref[1, :n]
    t = (jnp.dot(h_ref[...], wl_ref[...], preferred_element_type=jnp.float32)
         + b_ref[...] + ssum[:, :m] * iv_ref[...])
    mean = jnp.mean(t, axis=0, keepdims=True)
    var = jnp.mean((t - mean) ** 2, axis=0, keepdims=True)
    xn = (t - mean) / jnp.sqrt(var + EPS)
    hn = jnp.maximum(xn * g_ref[...] + be_ref[...], 0.0)
    t1 = jnp.maximum(
        jnp.dot(hn, w1_ref[...], preferred_element_type=jnp.float32)
        + b1_ref[...], 0.0)
    o_ref[...] = (jnp.dot(t1, w2_ref[...], preferred_element_type=jnp.float32)
                  + bb2_ref[...])

  return pl.pallas_call(
      body, out_shape=jax.ShapeDtypeStruct((n, 1), jnp.float32),
  )(h, wlT, b2, sp, invd, g2, be2, hw1T, hb1, hw2T, hb2)


def kernel(x, edge_index, batch, W0, b0, g0, be0, W1, b1, g1, be1,
           W2, b2, g2, be2, Hw1, Hb1, Hw2, Hb2):
  del batch
  n, d = x.shape
  hd = W0.shape[0]
  srct = edge_index[0].reshape(NW, -1, NB, C)
  dstt = edge_index[1].reshape(NW, -1, NB, C)
  idx3 = jnp.stack([srct, dstt], axis=2)

  w0l, w0r = W0[:, :d].T, W0[:, d:].T
  w1l, w1r = W1[:, :hd].T, W1[:, hd:].T
  w2l, w2r = W2[:, :hd].T, W2[:, hd:].T
  r2 = lambda v: v.reshape(1, -1)

  y0 = _tc_project(x, w0r, with_ones=True)
  s0 = _sc_aggregate(y0, idx3)
  h1, y1, invd = _tc_combine(x, w0l, r2(b0), s0, None, r2(g0), r2(be0), w1r)
  s1 = _sc_aggregate(y1, idx3)
  h2, y2 = _tc_combine(h1, w1l, r2(b1), s1, invd, r2(g1), r2(be1), w2r)
  s2 = _sc_aggregate(y2, idx3)
  out = _tc_final(h2, w2l, r2(b2), s2, invd, r2(g2), r2(be2),
                  Hw1.T, r2(Hb1), Hw2.T, r2(Hb2))
  return out
